# baseline (device time: 157164 ns/iter reference)
import jax
import jax.numpy as jnp
from jax import lax
from jax.experimental import pallas as pl
from jax.experimental.pallas import tpu as pltpu

N_DEV = 4
P = 320


def _a2a(xg, c2):
    n = xg.shape[1]

    def body(x_ref, c_ref, stg_ref, cnt_ref,
             dsend, drecv, csend, crecv):
        my = lax.axis_index("i")

        barrier = pltpu.get_barrier_semaphore()
        for k in range(1, N_DEV):
            nbr = lax.rem(my + k, N_DEV)
            pl.semaphore_signal(
                barrier, inc=1,
                device_id=(nbr,), device_id_type=pl.DeviceIdType.MESH,
            )
        pl.semaphore_wait(barrier, N_DEV - 1)

        sends = []
        for k in range(1, N_DEV):
            p = lax.rem(my + k, N_DEV)
            dr = pltpu.make_async_remote_copy(
                src_ref=x_ref.at[pl.ds(p * P, P), :],
                dst_ref=stg_ref.at[pl.ds(my * P, P), :],
                send_sem=dsend.at[k - 1],
                recv_sem=drecv.at[my],
                device_id=(p,),
                device_id_type=pl.DeviceIdType.MESH,
            )
            cr = pltpu.make_async_remote_copy(
                src_ref=c_ref,
                dst_ref=cnt_ref.at[pl.ds(my * 8, 8), :],
                send_sem=csend.at[k - 1],
                recv_sem=crecv.at[my],
                device_id=(p,),
                device_id_type=pl.DeviceIdType.MESH,
            )
            dr.start()
            cr.start()
            sends.append((dr, cr))

        stg_ref[pl.ds(my * P, P), :] = x_ref[pl.ds(my * P, P), :]
        cnt_ref[pl.ds(my * 8, 8), :] = c_ref[:, :]

        for dr, cr in sends:
            dr.wait_send()
            cr.wait_send()

        for k in range(1, N_DEV):
            r = lax.rem(my + N_DEV - k, N_DEV)
            pltpu.make_async_remote_copy(
                src_ref=x_ref.at[pl.ds(0, P), :],
                dst_ref=stg_ref.at[pl.ds(r * P, P), :],
                send_sem=dsend.at[k - 1],
                recv_sem=drecv.at[r],
                device_id=(r,),
                device_id_type=pl.DeviceIdType.MESH,
            ).wait_recv()
            pltpu.make_async_remote_copy(
                src_ref=c_ref,
                dst_ref=cnt_ref.at[pl.ds(r * 8, 8), :],
                send_sem=csend.at[k - 1],
                recv_sem=crecv.at[r],
                device_id=(r,),
                device_id_type=pl.DeviceIdType.MESH,
            ).wait_recv()

    return pl.pallas_call(
        body,
        out_shape=(
            jax.ShapeDtypeStruct((N_DEV * P, n), xg.dtype),
            jax.ShapeDtypeStruct((N_DEV * 8, 128), c2.dtype),
        ),
        in_specs=[
            pl.BlockSpec(memory_space=pltpu.VMEM),
            pl.BlockSpec(memory_space=pltpu.VMEM),
        ],
        out_specs=(
            pl.BlockSpec(memory_space=pltpu.VMEM),
            pl.BlockSpec(memory_space=pltpu.VMEM),
        ),
        scratch_shapes=[
            pltpu.SemaphoreType.DMA((N_DEV - 1,)),
            pltpu.SemaphoreType.DMA((N_DEV,)),
            pltpu.SemaphoreType.DMA((N_DEV - 1,)),
            pltpu.SemaphoreType.DMA((N_DEV,)),
        ],
        compiler_params=pltpu.CompilerParams(collective_id=0),
    )(xg, c2)


def kernel(x, dest):
    m = x.shape[0]

    order = jnp.argsort(dest, stable=True)
    xs = jnp.take(x, order, axis=0)

    tgt = jnp.arange(N_DEV, dtype=jnp.int32)
    cnts = jnp.sum(dest[None, :] == tgt[:, None], axis=1).astype(jnp.int32)
    starts = (jnp.cumsum(cnts) - cnts).astype(jnp.int32)
    c2 = jnp.zeros((8, 128), jnp.int32).at[0, :N_DEV].set(cnts)

    q = jnp.arange(N_DEV * P, dtype=jnp.int32)
    t_of_q = q // P
    w_of_q = q - t_of_q * P
    src_idx = jnp.clip(starts[t_of_q] + w_of_q, 0, m - 1)
    xg = jnp.take(xs, src_idx, axis=0)

    stg, cnt_all = _a2a(xg, c2)

    my = lax.axis_index("i")
    col = cnt_all[::8, :N_DEV][:, my]
    incl = jnp.cumsum(col)
    j = jnp.arange(m, dtype=jnp.int32)
    r = jnp.sum(j[:, None] >= incl[None, :], axis=1).astype(jnp.int32)
    within = j - (incl[r] - col[r]).astype(jnp.int32)
    return jnp.take(stg, r * P + within, axis=0)


# device time: 40993 ns/iter; 3.8339x vs baseline; 3.8339x over previous
import jax
import jax.numpy as jnp
from jax import lax
from jax.experimental import pallas as pl
from jax.experimental.pallas import tpu as pltpu

N_DEV = 4
P = 320


def _a2a(x, g, c2):
    m, n = x.shape

    def body(x_ref, g_ref, c_ref, stg_ref, cnt_ref,
             xg_ref, dsend, drecv, csend, crecv):
        my = lax.axis_index("i")

        barrier = pltpu.get_barrier_semaphore()
        for k in range(1, N_DEV):
            nbr = lax.rem(my + k, N_DEV)
            pl.semaphore_signal(
                barrier, inc=1,
                device_id=(nbr,), device_id_type=pl.DeviceIdType.MESH,
            )
        pl.semaphore_wait(barrier, N_DEV - 1)

        sends = []
        for k in range(1, N_DEV):
            p = lax.rem(my + k, N_DEV)
            xg_ref[pl.ds(p * P, P), :] = jnp.dot(
                g_ref[pl.ds(p * P, P), :], x_ref[:, :],
                preferred_element_type=jnp.float32,
                precision=lax.Precision.HIGHEST,
            )
            dr = pltpu.make_async_remote_copy(
                src_ref=xg_ref.at[pl.ds(p * P, P), :],
                dst_ref=stg_ref.at[pl.ds(my * P, P), :],
                send_sem=dsend.at[k - 1],
                recv_sem=drecv.at[my],
                device_id=(p,),
                device_id_type=pl.DeviceIdType.MESH,
            )
            cr = pltpu.make_async_remote_copy(
                src_ref=c_ref,
                dst_ref=cnt_ref.at[pl.ds(my * 8, 8), :],
                send_sem=csend.at[k - 1],
                recv_sem=crecv.at[my],
                device_id=(p,),
                device_id_type=pl.DeviceIdType.MESH,
            )
            dr.start()
            cr.start()
            sends.append((dr, cr))

        stg_ref[pl.ds(my * P, P), :] = jnp.dot(
            g_ref[pl.ds(my * P, P), :], x_ref[:, :],
            preferred_element_type=jnp.float32,
            precision=lax.Precision.HIGHEST,
        )
        cnt_ref[pl.ds(my * 8, 8), :] = c_ref[:, :]

        for dr, cr in sends:
            dr.wait_send()
            cr.wait_send()

        for k in range(1, N_DEV):
            r = lax.rem(my + N_DEV - k, N_DEV)
            pltpu.make_async_remote_copy(
                src_ref=xg_ref.at[pl.ds(0, P), :],
                dst_ref=stg_ref.at[pl.ds(r * P, P), :],
                send_sem=dsend.at[k - 1],
                recv_sem=drecv.at[r],
                device_id=(r,),
                device_id_type=pl.DeviceIdType.MESH,
            ).wait_recv()
            pltpu.make_async_remote_copy(
                src_ref=c_ref,
                dst_ref=cnt_ref.at[pl.ds(r * 8, 8), :],
                send_sem=csend.at[k - 1],
                recv_sem=crecv.at[r],
                device_id=(r,),
                device_id_type=pl.DeviceIdType.MESH,
            ).wait_recv()

    return pl.pallas_call(
        body,
        out_shape=(
            jax.ShapeDtypeStruct((N_DEV * P, n), x.dtype),
            jax.ShapeDtypeStruct((N_DEV * 8, 128), c2.dtype),
        ),
        in_specs=[
            pl.BlockSpec(memory_space=pltpu.VMEM),
            pl.BlockSpec(memory_space=pltpu.VMEM),
            pl.BlockSpec(memory_space=pltpu.VMEM),
        ],
        out_specs=(
            pl.BlockSpec(memory_space=pltpu.VMEM),
            pl.BlockSpec(memory_space=pltpu.VMEM),
        ),
        scratch_shapes=[
            pltpu.VMEM((N_DEV * P, n), x.dtype),
            pltpu.SemaphoreType.DMA((N_DEV - 1,)),
            pltpu.SemaphoreType.DMA((N_DEV,)),
            pltpu.SemaphoreType.DMA((N_DEV - 1,)),
            pltpu.SemaphoreType.DMA((N_DEV,)),
        ],
        compiler_params=pltpu.CompilerParams(collective_id=0),
    )(x, g, c2)


def kernel(x, dest):
    m = x.shape[0]

    dest = dest.astype(jnp.int32)
    tgt = jnp.arange(N_DEV, dtype=jnp.int32)
    masks = (dest[None, :] == tgt[:, None])
    cums = jnp.cumsum(masks.astype(jnp.int32), axis=1)
    rank = jnp.sum(jnp.where(masks, cums - 1, 0), axis=0).astype(jnp.int32)
    pos = dest * P + rank

    cnts = cums[:, -1].astype(jnp.int32)
    c2 = jnp.zeros((8, 128), jnp.int32).at[0, :N_DEV].set(cnts)

    q = jnp.arange(N_DEV * P, dtype=jnp.int32)
    g = (pos[None, :] == q[:, None]).astype(jnp.float32)

    stg, cnt_all = _a2a(x, g, c2)

    my = lax.axis_index("i")
    col = jnp.sum(
        cnt_all[::8, :N_DEV]
        * (jnp.arange(N_DEV, dtype=jnp.int32)[None, :] == my),
        axis=1,
    )
    o_in = (jnp.cumsum(col) - col).astype(jnp.int32)

    out = jnp.zeros((m, x.shape[1]), x.dtype)
    zpad = jnp.zeros((m - P, x.shape[1]), x.dtype)
    for r in range(N_DEV):
        seg = jnp.concatenate([stg[r * P:(r + 1) * P], zpad], axis=0)
        cat2 = jnp.concatenate([seg, seg], axis=0)
        rolled = lax.dynamic_slice(cat2, (m - o_in[r], 0), (m, x.shape[1]))
        out = out + rolled
    return out


# device time: 39450 ns/iter; 3.9839x vs baseline; 1.0391x over previous
import jax
import jax.numpy as jnp
from jax import lax
from jax.experimental import pallas as pl
from jax.experimental.pallas import tpu as pltpu

N_DEV = 4
P = 288


def _a2a(x, pos2, c2):
    m, n = x.shape

    def slot_dot(pos_ref, x_ref, p):
        q = lax.broadcasted_iota(jnp.int32, (P, m), 0) + p * P
        g = (q == pos_ref[0:1, :]).astype(jnp.float32)
        return jnp.dot(g, x_ref[:, :],
                       preferred_element_type=jnp.float32,
                       precision=lax.Precision.HIGHEST)

    def body(x_ref, pos_ref, c_ref, stg_ref, cnt_ref,
             xg_ref, dsend, drecv, csend, crecv):
        my = lax.axis_index("i")

        barrier = pltpu.get_barrier_semaphore()
        for k in range(1, N_DEV):
            nbr = lax.rem(my + k, N_DEV)
            pl.semaphore_signal(
                barrier, inc=1,
                device_id=(nbr,), device_id_type=pl.DeviceIdType.MESH,
            )
        pl.semaphore_wait(barrier, N_DEV - 1)

        sends = []
        for k in range(1, N_DEV):
            p = lax.rem(my + k, N_DEV)
            xg_ref[pl.ds(p * P, P), :] = slot_dot(pos_ref, x_ref, p)
            dr = pltpu.make_async_remote_copy(
                src_ref=xg_ref.at[pl.ds(p * P, P), :],
                dst_ref=stg_ref.at[pl.ds(my * P, P), :],
                send_sem=dsend.at[k - 1],
                recv_sem=drecv.at[my],
                device_id=(p,),
                device_id_type=pl.DeviceIdType.MESH,
            )
            cr = pltpu.make_async_remote_copy(
                src_ref=c_ref,
                dst_ref=cnt_ref.at[pl.ds(my * 8, 8), :],
                send_sem=csend.at[k - 1],
                recv_sem=crecv.at[my],
                device_id=(p,),
                device_id_type=pl.DeviceIdType.MESH,
            )
            dr.start()
            cr.start()
            sends.append((dr, cr))

        stg_ref[pl.ds(my * P, P), :] = slot_dot(pos_ref, x_ref, my)
        cnt_ref[pl.ds(my * 8, 8), :] = c_ref[:, :]

        for dr, cr in sends:
            dr.wait_send()
            cr.wait_send()

        for k in range(1, N_DEV):
            r = lax.rem(my + N_DEV - k, N_DEV)
            pltpu.make_async_remote_copy(
                src_ref=xg_ref.at[pl.ds(0, P), :],
                dst_ref=stg_ref.at[pl.ds(r * P, P), :],
                send_sem=dsend.at[k - 1],
                recv_sem=drecv.at[r],
                device_id=(r,),
                device_id_type=pl.DeviceIdType.MESH,
            ).wait_recv()
            pltpu.make_async_remote_copy(
                src_ref=c_ref,
                dst_ref=cnt_ref.at[pl.ds(r * 8, 8), :],
                send_sem=csend.at[k - 1],
                recv_sem=crecv.at[r],
                device_id=(r,),
                device_id_type=pl.DeviceIdType.MESH,
            ).wait_recv()

    return pl.pallas_call(
        body,
        out_shape=(
            jax.ShapeDtypeStruct((N_DEV * P, n), x.dtype),
            jax.ShapeDtypeStruct((N_DEV * 8, 128), c2.dtype),
        ),
        in_specs=[
            pl.BlockSpec(memory_space=pltpu.VMEM),
            pl.BlockSpec(memory_space=pltpu.VMEM),
            pl.BlockSpec(memory_space=pltpu.VMEM),
        ],
        out_specs=(
            pl.BlockSpec(memory_space=pltpu.VMEM),
            pl.BlockSpec(memory_space=pltpu.VMEM),
        ),
        scratch_shapes=[
            pltpu.VMEM((N_DEV * P, n), x.dtype),
            pltpu.SemaphoreType.DMA((N_DEV - 1,)),
            pltpu.SemaphoreType.DMA((N_DEV,)),
            pltpu.SemaphoreType.DMA((N_DEV - 1,)),
            pltpu.SemaphoreType.DMA((N_DEV,)),
        ],
        compiler_params=pltpu.CompilerParams(collective_id=0),
    )(x, pos2, c2)


def kernel(x, dest):
    m = x.shape[0]

    dest = dest.astype(jnp.int32)
    tgt = jnp.arange(N_DEV, dtype=jnp.int32)
    masks = (dest[None, :] == tgt[:, None])
    cums = jnp.cumsum(masks.astype(jnp.int32), axis=1)
    rank = jnp.sum(jnp.where(masks, cums - 1, 0), axis=0).astype(jnp.int32)
    pos2 = (dest * P + rank).reshape(1, m)

    cnts = cums[:, -1].astype(jnp.int32)
    c2 = jnp.zeros((8, 128), jnp.int32).at[0, :N_DEV].set(cnts)

    stg, cnt_all = _a2a(x, pos2, c2)

    my = lax.axis_index("i")
    col = jnp.sum(
        cnt_all[::8, :N_DEV]
        * (jnp.arange(N_DEV, dtype=jnp.int32)[None, :] == my),
        axis=1,
    )
    o_in = (jnp.cumsum(col) - col).astype(jnp.int32)

    out = jnp.zeros((m, x.shape[1]), x.dtype)
    zpad = jnp.zeros((m - P, x.shape[1]), x.dtype)
    for r in range(N_DEV):
        seg = jnp.concatenate([stg[r * P:(r + 1) * P], zpad], axis=0)
        cat2 = jnp.concatenate([seg, seg], axis=0)
        rolled = lax.dynamic_slice(cat2, (m - o_in[r], 0), (m, x.shape[1]))
        out = out + rolled
    return out


# device time: 34355 ns/iter; 4.5747x vs baseline; 1.1483x over previous
import jax
import jax.numpy as jnp
from jax import lax
from jax.experimental import pallas as pl
from jax.experimental.pallas import tpu as pltpu

N_DEV = 4
P = 288


def _count_allgather(c2):
    def body(c_ref, cnt_ref, csend, crecv):
        my = lax.axis_index("i")

        barrier = pltpu.get_barrier_semaphore()
        for k in range(1, N_DEV):
            nbr = lax.rem(my + k, N_DEV)
            pl.semaphore_signal(
                barrier, inc=1,
                device_id=(nbr,), device_id_type=pl.DeviceIdType.MESH,
            )
        pl.semaphore_wait(barrier, N_DEV - 1)

        sends = []
        for k in range(1, N_DEV):
            p = lax.rem(my + k, N_DEV)
            cr = pltpu.make_async_remote_copy(
                src_ref=c_ref,
                dst_ref=cnt_ref.at[pl.ds(my * 8, 8), :],
                send_sem=csend.at[k - 1],
                recv_sem=crecv.at[my],
                device_id=(p,),
                device_id_type=pl.DeviceIdType.MESH,
            )
            cr.start()
            sends.append(cr)

        cnt_ref[pl.ds(my * 8, 8), :] = c_ref[:, :]
        for cr in sends:
            cr.wait_send()

        for k in range(1, N_DEV):
            r = lax.rem(my + N_DEV - k, N_DEV)
            pltpu.make_async_remote_copy(
                src_ref=c_ref,
                dst_ref=cnt_ref.at[pl.ds(r * 8, 8), :],
                send_sem=csend.at[k - 1],
                recv_sem=crecv.at[r],
                device_id=(r,),
                device_id_type=pl.DeviceIdType.MESH,
            ).wait_recv()

    return pl.pallas_call(
        body,
        out_shape=jax.ShapeDtypeStruct((N_DEV * 8, 128), c2.dtype),
        in_specs=[pl.BlockSpec(memory_space=pltpu.VMEM)],
        out_specs=pl.BlockSpec(memory_space=pltpu.VMEM),
        scratch_shapes=[
            pltpu.SemaphoreType.DMA((N_DEV - 1,)),
            pltpu.SemaphoreType.DMA((N_DEV,)),
        ],
        compiler_params=pltpu.CompilerParams(collective_id=0),
    )(c2)


def _a2a(x, pos2, o_in):
    m, n = x.shape

    def slot_dot(pos_ref, x_ref, p):
        q = lax.broadcasted_iota(jnp.int32, (P, m), 0) + p * P
        g = (q == pos_ref[0:1, :]).astype(jnp.float32)
        return jnp.dot(g, x_ref[:, :],
                       preferred_element_type=jnp.float32,
                       precision=lax.Precision.HIGHEST)

    def body(x_ref, pos_ref, o_ref, out_ref, xg_ref, stg_ref,
             dsend, drecv):
        my = lax.axis_index("i")

        barrier = pltpu.get_barrier_semaphore()
        for k in range(1, N_DEV):
            nbr = lax.rem(my + k, N_DEV)
            pl.semaphore_signal(
                barrier, inc=1,
                device_id=(nbr,), device_id_type=pl.DeviceIdType.MESH,
            )
        pl.semaphore_wait(barrier, N_DEV - 1)

        sends = []
        for k in range(1, N_DEV):
            p = lax.rem(my + k, N_DEV)
            xg_ref[pl.ds(p * P, P), :] = slot_dot(pos_ref, x_ref, p)
            dr = pltpu.make_async_remote_copy(
                src_ref=xg_ref.at[pl.ds(p * P, P), :],
                dst_ref=stg_ref.at[pl.ds(my * P, P), :],
                send_sem=dsend.at[k - 1],
                recv_sem=drecv.at[my],
                device_id=(p,),
                device_id_type=pl.DeviceIdType.MESH,
            )
            dr.start()
            sends.append(dr)

        zpad = jnp.zeros((m - P, n), jnp.float32)
        own = jnp.concatenate([slot_dot(pos_ref, x_ref, my), zpad], axis=0)
        out_ref[:, :] = pltpu.roll(own, o_ref[my], 0)

        for dr in sends:
            dr.wait_send()

        for k in range(1, N_DEV):
            r = lax.rem(my + N_DEV - k, N_DEV)
            pltpu.make_async_remote_copy(
                src_ref=xg_ref.at[pl.ds(0, P), :],
                dst_ref=stg_ref.at[pl.ds(r * P, P), :],
                send_sem=dsend.at[k - 1],
                recv_sem=drecv.at[r],
                device_id=(r,),
                device_id_type=pl.DeviceIdType.MESH,
            ).wait_recv()
            seg = jnp.concatenate(
                [stg_ref[pl.ds(r * P, P), :], zpad], axis=0)
            out_ref[:, :] += pltpu.roll(seg, o_ref[r], 0)

    return pl.pallas_call(
        body,
        out_shape=jax.ShapeDtypeStruct((m, n), x.dtype),
        in_specs=[
            pl.BlockSpec(memory_space=pltpu.VMEM),
            pl.BlockSpec(memory_space=pltpu.VMEM),
            pl.BlockSpec(memory_space=pltpu.SMEM),
        ],
        out_specs=pl.BlockSpec(memory_space=pltpu.VMEM),
        scratch_shapes=[
            pltpu.VMEM((N_DEV * P, n), x.dtype),
            pltpu.VMEM((N_DEV * P, n), x.dtype),
            pltpu.SemaphoreType.DMA((N_DEV - 1,)),
            pltpu.SemaphoreType.DMA((N_DEV,)),
        ],
        compiler_params=pltpu.CompilerParams(collective_id=1),
    )(x, pos2, o_in)


def kernel(x, dest):
    m = x.shape[0]

    dest = dest.astype(jnp.int32)
    tgt = jnp.arange(N_DEV, dtype=jnp.int32)
    masks = (dest[None, :] == tgt[:, None])
    cums = jnp.cumsum(masks.astype(jnp.int32), axis=1)
    rank = jnp.sum(jnp.where(masks, cums - 1, 0), axis=0).astype(jnp.int32)
    pos2 = (dest * P + rank).reshape(1, m)

    cnts = cums[:, -1].astype(jnp.int32)
    c2 = jnp.zeros((8, 128), jnp.int32).at[0, :N_DEV].set(cnts)

    cnt_all = _count_allgather(c2)

    my = lax.axis_index("i")
    col = jnp.sum(
        cnt_all[::8, :N_DEV]
        * (jnp.arange(N_DEV, dtype=jnp.int32)[None, :] == my),
        axis=1,
    )
    o_in = (jnp.cumsum(col) - col).astype(jnp.int32)

    return _a2a(x, pos2, o_in)
